# Initial kernel scaffold; baseline (speedup 1.0000x reference)
import jax
import jax.numpy as jnp
from jax.experimental import pallas as pl


def kernel(x, table):
    raise NotImplementedError("write your pallas kernel here")


# speedup vs baseline: 1.0589x; 1.0589x over previous
import math

import jax
import jax.numpy as jnp
from jax import lax
from jax.experimental import pallas as pl
from jax.experimental.pallas import tpu as pltpu
from jax.experimental.pallas import tpu_sc as plsc

VOCAB_D = 512
SCALE = math.sqrt(VOCAB_D)
LANES = 16

NUM_CORES = 2
NUM_SUBCORES = 16
NW = NUM_CORES * NUM_SUBCORES

B_TOTAL = 4096 * 50
B_PER_W = B_TOTAL // NW
CHUNK = 64
NCHUNK = B_PER_W // CHUNK
NBUF = 2


def _sc_body(table_hbm, idx_hbm, out_hbm, idx_v, buf0, buf1, sg0, sg1, so0, so1):
  cid = lax.axis_index("c")
  sid = lax.axis_index("s")
  wid = sid * NUM_CORES + cid

  bufs = (buf0, buf1)
  gsems = (sg0, sg1)
  osems = (so0, so1)

  pltpu.sync_copy(idx_hbm.at[wid], idx_v)

  def gcp(c, b):
    return pltpu.make_async_copy(table_hbm.at[idx_v.at[c]], bufs[b], gsems[b])

  def ocp(c, b):
    base = wid * B_PER_W + c * CHUNK
    return pltpu.make_async_copy(
        bufs[b], out_hbm.at[pl.ds(base, CHUNK)], osems[b])

  for b in range(NBUF):
    gcp(b, b).start()

  def do_round(p, start_next):
    for b in range(NBUF):
      g = p * NBUF + b
      gcp(g, b).wait()

      buf = bufs[b]

      @plsc.parallel_loop(0, CHUNK)
      def _(r):
        for j in range(VOCAB_D // LANES):
          sl = pl.ds(j * LANES, LANES)
          buf[r, sl] = buf[r, sl] * SCALE

      ocp(g, b).start()
      if start_next:
        ocp(g, b).wait()
        gcp(g + NBUF, b).start()

  nrounds = NCHUNK // NBUF

  def loop_body(p, carry):
    do_round(p, True)
    return carry

  lax.fori_loop(0, nrounds - 1, loop_body, jnp.int32(0))
  do_round(nrounds - 1, False)

  for b in range(NBUF):
    ocp(NCHUNK - NBUF + b, b).wait()


def _make_sc_call():
  mesh = plsc.VectorSubcoreMesh(core_axis_name="c", subcore_axis_name="s")
  return pl.kernel(
      _sc_body,
      out_type=jax.ShapeDtypeStruct((B_TOTAL, VOCAB_D), jnp.float32),
      mesh=mesh,
      scratch_types=[
          pltpu.VMEM((NCHUNK, CHUNK), jnp.int32),
          pltpu.VMEM((CHUNK, VOCAB_D), jnp.float32),
          pltpu.VMEM((CHUNK, VOCAB_D), jnp.float32),
          pltpu.SemaphoreType.DMA,
          pltpu.SemaphoreType.DMA,
          pltpu.SemaphoreType.DMA,
          pltpu.SemaphoreType.DMA,
      ],
      name="embedding_gather_scale_sc",
  )


def kernel(x, table):
  b, s = x.shape
  idx = x.reshape(NW, NCHUNK, CHUNK).astype(jnp.int32)
  out = _make_sc_call()(table, idx)
  return out.reshape(b, s, VOCAB_D)
